# baseline (speedup 1.0000x reference)
import functools

import jax
import jax.numpy as jnp
from jax import lax
from jax.experimental import pallas as pl
from jax.experimental.pallas import tpu as pltpu
from jax.experimental.pallas import tpu_sc as plsc

N = 10000
E = 320000
D = 128
G = 64

NW = 32
EPW = E // NW
CH = 32
NBUF = 8
LOOK = NBUF // 2
NCHUNK = 320
EPW_PAD = NCHUNK * CH
NSEG = 8
NHALF = NCHUNK // NSEG
NGRP = NHALF // NBUF
NPAD = 10112
RPT = NPAD // 16
BLK = 1000
NBLK = N // BLK


_mesh = plsc.VectorSubcoreMesh(core_axis_name="c", subcore_axis_name="s")


@functools.partial(
    pl.kernel,
    out_type=jax.ShapeDtypeStruct((2, NPAD, D), jnp.float32),
    mesh=_mesh,
    scratch_types=[
        pltpu.VMEM((NHALF, CH), jnp.int32),
        pltpu.VMEM((NHALF, CH), jnp.int32),
    ] + [pltpu.VMEM((CH, D), jnp.float32) for _ in range(NBUF)]
    + [
        pltpu.VMEM_SHARED((NPAD, D), jnp.float32),
    ] + [pltpu.SemaphoreType.DMA for _ in range(NBUF)],
)
def _sc_scatter(x_hbm, src_hbm, dst_hbm, zeros_hbm, out_hbm,
                src_v, dst_v, *rest):
    bufs = rest[:NBUF]
    agg_s = rest[NBUF]
    sems = rest[NBUF + 1:]
    c = lax.axis_index("c")
    s = lax.axis_index("s")
    w = c * 16 + s
    off = s * RPT
    pltpu.sync_copy(zeros_hbm.at[pl.ds(off, RPT)], agg_s.at[pl.ds(off, RPT)])
    plsc.subcore_barrier()

    def gather(j, b):
        pltpu.async_copy(x_hbm.at[src_v.at[jnp.minimum(j, NHALF - 1)]],
                         bufs[b], sems[b])

    def gwait(b):
        pltpu.make_async_copy(x_hbm.at[src_v.at[0]], bufs[b], sems[b]).wait()

    def scatter(j, b):
        pltpu.async_copy(bufs[b], agg_s.at[dst_v.at[j]], sems[b], add=True)

    def swait(b):
        pltpu.make_async_copy(bufs[b], agg_s.at[dst_v.at[0]], sems[b]).wait()

    def group(j0, first):
        for k in range(NBUF):
            j = j0 + k
            gwait(k)
            scatter(j, k)
            b2 = (k + LOOK) % NBUF
            if not (first and k < LOOK):
                swait(b2)
            gather(j + LOOK, b2)

    def body(grp, carry):
        group(grp * NBUF, False)
        return carry

    for seg in range(NSEG):
        pltpu.sync_copy(src_hbm.at[w, pl.ds(seg * NHALF, NHALF)], src_v)
        pltpu.sync_copy(dst_hbm.at[w, pl.ds(seg * NHALF, NHALF)], dst_v)
        for k in range(LOOK):
            gather(k, k)
        group(0, True)
        lax.fori_loop(1, NGRP, body, 0)
        for k in range(LOOK):
            gwait(k)
            swait(k + LOOK)
    plsc.subcore_barrier()
    pltpu.sync_copy(agg_s.at[pl.ds(off, RPT)], out_hbm.at[c, pl.ds(off, RPT)])


def _tc_layer_body(x_ref, p0_ref, p1_ref, w1_ref, sc_ref, sh_ref,
                   w2_ref, b2_ref, o_ref):
    h = (x_ref[...]
         + p0_ref[...].reshape(BLK, D)
         + p1_ref[...].reshape(BLK, D))
    y = jnp.dot(h, w1_ref[...], preferred_element_type=jnp.float32)
    y = jnp.maximum(y * sc_ref[...] + sh_ref[...], 0.0)
    z = jnp.dot(y, w2_ref[...], preferred_element_type=jnp.float32)
    o_ref[...] = jnp.maximum(z + b2_ref[...], 0.0)


_tc_layer = pl.pallas_call(
    _tc_layer_body,
    grid=(NBLK,),
    in_specs=[
        pl.BlockSpec((BLK, D), lambda i: (i, 0)),
        pl.BlockSpec((1, BLK, D), lambda i: (0, i, 0)),
        pl.BlockSpec((1, BLK, D), lambda i: (1, i, 0)),
        pl.BlockSpec((D, D), lambda i: (0, 0)),
        pl.BlockSpec((1, D), lambda i: (0, 0)),
        pl.BlockSpec((1, D), lambda i: (0, 0)),
        pl.BlockSpec((D, D), lambda i: (0, 0)),
        pl.BlockSpec((1, D), lambda i: (0, 0)),
    ],
    out_specs=pl.BlockSpec((BLK, D), lambda i: (i, 0)),
    out_shape=jax.ShapeDtypeStruct((N, D), jnp.float32),
)


def _tc_pool_body(bt_ref, x_ref, w_ref, b_ref, o_ref, acc_ref):
    i = pl.program_id(0)

    @pl.when(i == 0)
    def _():
        acc_ref[...] = jnp.zeros_like(acc_ref)

    ids = lax.broadcasted_iota(jnp.int32, (G, BLK), 0)
    m = (bt_ref[...].reshape(1, BLK) == ids).astype(jnp.float32)
    acc_ref[...] += jnp.dot(m, x_ref[...], preferred_element_type=jnp.float32)

    @pl.when(i == NBLK - 1)
    def _():
        p = jnp.dot(acc_ref[...], w_ref[...], preferred_element_type=jnp.float32)
        o_ref[...] = jnp.maximum(p + b_ref[...], 0.0)


_tc_pool = pl.pallas_call(
    _tc_pool_body,
    grid=(NBLK,),
    in_specs=[
        pl.BlockSpec((1, 1, BLK), lambda i: (i, 0, 0)),
        pl.BlockSpec((BLK, D), lambda i: (i, 0)),
        pl.BlockSpec((D, D), lambda i: (0, 0)),
        pl.BlockSpec((1, D), lambda i: (0, 0)),
    ],
    out_specs=pl.BlockSpec((G, D), lambda i: (0, 0)),
    out_shape=jax.ShapeDtypeStruct((G, D), jnp.float32),
    scratch_shapes=[pltpu.VMEM((G, D), jnp.float32)],
)


def kernel(x, edge_index, batch,
           c1_w1, c1_b1, c1_g, c1_be, c1_w2, c1_b2,
           c2_w1, c2_b1, c2_g, c2_be, c2_w2, c2_b2,
           c3_w1, c3_b1, c3_g, c3_be, c3_w2, c3_b2,
           c4_w1, c4_b1, c4_g, c4_be, c4_w2, c4_b2,
           c5_w1, c5_b1, c5_g, c5_be, c5_w2, c5_b2,
           lin1_w, lin1_b):
    pad = EPW_PAD * NW - E
    pad_src = jnp.arange(pad, dtype=jnp.int32) % N
    pad_dst = N + jnp.arange(pad, dtype=jnp.int32) % (NPAD - N)
    src = jnp.concatenate([edge_index[0], pad_src])
    dst = jnp.concatenate([edge_index[1], pad_dst])
    src_r = src.reshape(NW, NCHUNK, CH)
    dst_r = dst.reshape(NW, NCHUNK, CH)
    zeros = jnp.zeros((NPAD, D), jnp.float32)
    batch_r = batch.reshape(NBLK, 1, BLK)

    inv = 1.0 / jnp.sqrt(jnp.float32(1.0 + 1e-5))
    layers = [
        (c1_w1, c1_b1, c1_g, c1_be, c1_w2, c1_b2),
        (c2_w1, c2_b1, c2_g, c2_be, c2_w2, c2_b2),
        (c3_w1, c3_b1, c3_g, c3_be, c3_w2, c3_b2),
        (c4_w1, c4_b1, c4_g, c4_be, c4_w2, c4_b2),
        (c5_w1, c5_b1, c5_g, c5_be, c5_w2, c5_b2),
    ]
    for w1, b1, g, be, w2, b2 in layers:
        part = _sc_scatter(x, src_r, dst_r, zeros)
        sc = (g * inv).reshape(1, D)
        sh = (b1 * g * inv + be).reshape(1, D)
        x = _tc_layer(x, part, part, w1, sc, sh, w2, b2.reshape(1, D))
    return _tc_pool(batch_r, x, lin1_w, lin1_b.reshape(1, D))


# speedup vs baseline: 1.0651x; 1.0651x over previous
import functools

import jax
import jax.numpy as jnp
from jax import lax
from jax.experimental import pallas as pl
from jax.experimental.pallas import tpu as pltpu
from jax.experimental.pallas import tpu_sc as plsc

N = 10000
E = 320000
D = 128
G = 64

NW = 32
EPW = E // NW
CH = 64
NBUF = 4
LOOK = NBUF // 2
NCHUNK = 160
EPW_PAD = NCHUNK * CH
NSEG = 4
NHALF = NCHUNK // NSEG
NGRP = NHALF // NBUF
NPAD = 10112
RPT = NPAD // 16
BLK = 1000
NBLK = N // BLK


_mesh = plsc.VectorSubcoreMesh(core_axis_name="c", subcore_axis_name="s")


@functools.partial(
    pl.kernel,
    out_type=jax.ShapeDtypeStruct((2, NPAD, D), jnp.float32),
    mesh=_mesh,
    scratch_types=[
        pltpu.VMEM((NHALF, CH), jnp.int32),
        pltpu.VMEM((NHALF, CH), jnp.int32),
    ] + [pltpu.VMEM((CH, D), jnp.float32) for _ in range(NBUF)]
    + [
        pltpu.VMEM_SHARED((NPAD, D), jnp.float32),
    ] + [pltpu.SemaphoreType.DMA for _ in range(NBUF)],
)
def _sc_scatter(x_hbm, src_hbm, dst_hbm, zeros_hbm, out_hbm,
                src_v, dst_v, *rest):
    bufs = rest[:NBUF]
    agg_s = rest[NBUF]
    sems = rest[NBUF + 1:]
    c = lax.axis_index("c")
    s = lax.axis_index("s")
    w = c * 16 + s
    off = s * RPT

    def gather(j, b):
        pltpu.async_copy(x_hbm.at[src_v.at[jnp.minimum(j, NHALF - 1)]],
                         bufs[b], sems[b])

    def gwait(b):
        pltpu.make_async_copy(x_hbm.at[src_v.at[0]], bufs[b], sems[b]).wait()

    def scatter(j, b):
        pltpu.async_copy(bufs[b], agg_s.at[dst_v.at[j]], sems[b], add=True)

    def swait(b):
        pltpu.make_async_copy(bufs[b], agg_s.at[dst_v.at[0]], sems[b]).wait()

    def group(j0, first):
        for k in range(NBUF):
            j = j0 + k
            gwait(k)
            scatter(j, k)
            b2 = (k + LOOK) % NBUF
            if not (first and k < LOOK):
                swait(b2)
            gather(j + LOOK, b2)

    def body(grp, carry):
        group(grp * NBUF, False)
        return carry

    def stage(seg):
        pltpu.sync_copy(src_hbm.at[w, pl.ds(seg * NHALF, NHALF)], src_v)
        pltpu.sync_copy(dst_hbm.at[w, pl.ds(seg * NHALF, NHALF)], dst_v)
        for k in range(LOOK):
            gather(k, k)

    stage(0)
    pltpu.sync_copy(zeros_hbm.at[pl.ds(off, RPT)], agg_s.at[pl.ds(off, RPT)])
    plsc.subcore_barrier()

    for seg in range(NSEG):
        if seg > 0:
            stage(seg)
        group(0, True)
        lax.fori_loop(1, NGRP, body, 0)
        for k in range(LOOK):
            gwait(k)
            swait(k + LOOK)
    plsc.subcore_barrier()
    pltpu.sync_copy(agg_s.at[pl.ds(off, RPT)], out_hbm.at[c, pl.ds(off, RPT)])


def _tc_layer_body(x_ref, p0_ref, p1_ref, w1_ref, sc_ref, sh_ref,
                   w2_ref, b2_ref, o_ref):
    h = (x_ref[...]
         + p0_ref[...].reshape(BLK, D)
         + p1_ref[...].reshape(BLK, D))
    y = jnp.dot(h, w1_ref[...], preferred_element_type=jnp.float32)
    y = jnp.maximum(y * sc_ref[...] + sh_ref[...], 0.0)
    z = jnp.dot(y, w2_ref[...], preferred_element_type=jnp.float32)
    o_ref[...] = jnp.maximum(z + b2_ref[...], 0.0)


_tc_layer = pl.pallas_call(
    _tc_layer_body,
    grid=(NBLK,),
    in_specs=[
        pl.BlockSpec((BLK, D), lambda i: (i, 0)),
        pl.BlockSpec((1, BLK, D), lambda i: (0, i, 0)),
        pl.BlockSpec((1, BLK, D), lambda i: (1, i, 0)),
        pl.BlockSpec((D, D), lambda i: (0, 0)),
        pl.BlockSpec((1, D), lambda i: (0, 0)),
        pl.BlockSpec((1, D), lambda i: (0, 0)),
        pl.BlockSpec((D, D), lambda i: (0, 0)),
        pl.BlockSpec((1, D), lambda i: (0, 0)),
    ],
    out_specs=pl.BlockSpec((BLK, D), lambda i: (i, 0)),
    out_shape=jax.ShapeDtypeStruct((N, D), jnp.float32),
)


def _tc_pool_body(bt_ref, x_ref, w_ref, b_ref, o_ref, acc_ref):
    i = pl.program_id(0)

    @pl.when(i == 0)
    def _():
        acc_ref[...] = jnp.zeros_like(acc_ref)

    ids = lax.broadcasted_iota(jnp.int32, (G, BLK), 0)
    m = (bt_ref[...].reshape(1, BLK) == ids).astype(jnp.float32)
    acc_ref[...] += jnp.dot(m, x_ref[...], preferred_element_type=jnp.float32)

    @pl.when(i == NBLK - 1)
    def _():
        p = jnp.dot(acc_ref[...], w_ref[...], preferred_element_type=jnp.float32)
        o_ref[...] = jnp.maximum(p + b_ref[...], 0.0)


_tc_pool = pl.pallas_call(
    _tc_pool_body,
    grid=(NBLK,),
    in_specs=[
        pl.BlockSpec((1, 1, BLK), lambda i: (i, 0, 0)),
        pl.BlockSpec((BLK, D), lambda i: (i, 0)),
        pl.BlockSpec((D, D), lambda i: (0, 0)),
        pl.BlockSpec((1, D), lambda i: (0, 0)),
    ],
    out_specs=pl.BlockSpec((G, D), lambda i: (0, 0)),
    out_shape=jax.ShapeDtypeStruct((G, D), jnp.float32),
    scratch_shapes=[pltpu.VMEM((G, D), jnp.float32)],
)


def kernel(x, edge_index, batch,
           c1_w1, c1_b1, c1_g, c1_be, c1_w2, c1_b2,
           c2_w1, c2_b1, c2_g, c2_be, c2_w2, c2_b2,
           c3_w1, c3_b1, c3_g, c3_be, c3_w2, c3_b2,
           c4_w1, c4_b1, c4_g, c4_be, c4_w2, c4_b2,
           c5_w1, c5_b1, c5_g, c5_be, c5_w2, c5_b2,
           lin1_w, lin1_b):
    pad = EPW_PAD * NW - E
    pad_src = jnp.arange(pad, dtype=jnp.int32) % N
    pad_dst = N + jnp.arange(pad, dtype=jnp.int32) % (NPAD - N)
    src = jnp.concatenate([edge_index[0], pad_src])
    dst = jnp.concatenate([edge_index[1], pad_dst])
    src_r = src.reshape(NW, NCHUNK, CH)
    dst_r = dst.reshape(NW, NCHUNK, CH)
    zeros = jnp.zeros((NPAD, D), jnp.float32)
    batch_r = batch.reshape(NBLK, 1, BLK)

    inv = 1.0 / jnp.sqrt(jnp.float32(1.0 + 1e-5))
    layers = [
        (c1_w1, c1_b1, c1_g, c1_be, c1_w2, c1_b2),
        (c2_w1, c2_b1, c2_g, c2_be, c2_w2, c2_b2),
        (c3_w1, c3_b1, c3_g, c3_be, c3_w2, c3_b2),
        (c4_w1, c4_b1, c4_g, c4_be, c4_w2, c4_b2),
        (c5_w1, c5_b1, c5_g, c5_be, c5_w2, c5_b2),
    ]
    for w1, b1, g, be, w2, b2 in layers:
        part = _sc_scatter(x, src_r, dst_r, zeros)
        sc = (g * inv).reshape(1, D)
        sh = (b1 * g * inv + be).reshape(1, D)
        x = _tc_layer(x, part, part, w1, sc, sh, w2, b2.reshape(1, D))
    return _tc_pool(batch_r, x, lin1_w, lin1_b.reshape(1, D))


# speedup vs baseline: 1.1915x; 1.1186x over previous
import functools

import jax
import jax.numpy as jnp
from jax import lax
from jax.experimental import pallas as pl
from jax.experimental.pallas import tpu as pltpu
from jax.experimental.pallas import tpu_sc as plsc

N = 10000
E = 320000
D = 128
G = 64

NW = 32
EPW = E // NW
CH = 64
NBUF = 4
LOOK = NBUF // 2
NCHUNK = 160
EPW_PAD = NCHUNK * CH
NSEG = 4
NHALF = NCHUNK // NSEG
NGRP = NHALF // NBUF
NPAD = 10112
RPT = NPAD // 16
BLK = 1000
NBLK = N // BLK


_mesh = plsc.VectorSubcoreMesh(core_axis_name="c", subcore_axis_name="s")


@functools.partial(
    pl.kernel,
    out_type=jax.ShapeDtypeStruct((2, NPAD, D), jnp.float32),
    mesh=_mesh,
    scratch_types=[
        pltpu.VMEM((NHALF, CH), jnp.int32),
        pltpu.VMEM((NHALF, CH), jnp.int32),
    ] + [pltpu.VMEM((CH, D), jnp.float32) for _ in range(NBUF)]
    + [
        pltpu.VMEM_SHARED((NPAD, D), jnp.float32),
    ] + [pltpu.SemaphoreType.DMA for _ in range(NBUF)],
)
def _sc_scatter(x_hbm, src_hbm, dst_hbm, zeros_hbm, out_hbm,
                src_v, dst_v, *rest):
    bufs = rest[:NBUF]
    agg_s = rest[NBUF]
    sems = rest[NBUF + 1:]
    c = lax.axis_index("c")
    s = lax.axis_index("s")
    w = c * 16 + s
    off = s * RPT

    def gather(j, b):
        pltpu.async_copy(x_hbm.at[src_v.at[jnp.minimum(j, NHALF - 1)]],
                         bufs[b], sems[b])

    def gwait(b):
        pltpu.make_async_copy(x_hbm.at[src_v.at[0]], bufs[b], sems[b]).wait()

    def scatter(j, b):
        pltpu.async_copy(bufs[b], agg_s.at[dst_v.at[j]], sems[b], add=True)

    def swait(b):
        pltpu.make_async_copy(bufs[b], agg_s.at[dst_v.at[0]], sems[b]).wait()

    def group(j0, first):
        for k in range(NBUF):
            j = j0 + k
            b2 = (k + LOOK) % NBUF
            if not (first and k < LOOK):
                swait(b2)
            gather(j + LOOK, b2)
            gwait(k)
            scatter(j, k)

    def body(grp, carry):
        group(grp * NBUF, False)
        return carry

    def stage(seg):
        pltpu.sync_copy(src_hbm.at[w, pl.ds(seg * NHALF, NHALF)], src_v)
        pltpu.sync_copy(dst_hbm.at[w, pl.ds(seg * NHALF, NHALF)], dst_v)
        for k in range(LOOK):
            gather(k, k)

    stage(0)
    pltpu.sync_copy(zeros_hbm.at[pl.ds(off, RPT)], agg_s.at[pl.ds(off, RPT)])
    plsc.subcore_barrier()

    for seg in range(NSEG):
        if seg > 0:
            stage(seg)
        group(0, True)
        lax.fori_loop(1, NGRP, body, 0)
        for k in range(LOOK):
            gwait(k)
            swait(k + LOOK)
    plsc.subcore_barrier()
    pltpu.sync_copy(agg_s.at[pl.ds(off, RPT)], out_hbm.at[c, pl.ds(off, RPT)])


def _tc_layer_body(x_ref, p0_ref, p1_ref, w1_ref, sc_ref, sh_ref,
                   w2_ref, b2_ref, o_ref):
    h = (x_ref[...]
         + p0_ref[...].reshape(BLK, D)
         + p1_ref[...].reshape(BLK, D))
    y = jnp.dot(h, w1_ref[...], preferred_element_type=jnp.float32)
    y = jnp.maximum(y * sc_ref[...] + sh_ref[...], 0.0)
    z = jnp.dot(y, w2_ref[...], preferred_element_type=jnp.float32)
    o_ref[...] = jnp.maximum(z + b2_ref[...], 0.0)


_tc_layer = pl.pallas_call(
    _tc_layer_body,
    grid=(NBLK,),
    in_specs=[
        pl.BlockSpec((BLK, D), lambda i: (i, 0)),
        pl.BlockSpec((1, BLK, D), lambda i: (0, i, 0)),
        pl.BlockSpec((1, BLK, D), lambda i: (1, i, 0)),
        pl.BlockSpec((D, D), lambda i: (0, 0)),
        pl.BlockSpec((1, D), lambda i: (0, 0)),
        pl.BlockSpec((1, D), lambda i: (0, 0)),
        pl.BlockSpec((D, D), lambda i: (0, 0)),
        pl.BlockSpec((1, D), lambda i: (0, 0)),
    ],
    out_specs=pl.BlockSpec((BLK, D), lambda i: (i, 0)),
    out_shape=jax.ShapeDtypeStruct((N, D), jnp.float32),
)


def _tc_pool_body(bt_ref, x_ref, w_ref, b_ref, o_ref, acc_ref):
    i = pl.program_id(0)

    @pl.when(i == 0)
    def _():
        acc_ref[...] = jnp.zeros_like(acc_ref)

    ids = lax.broadcasted_iota(jnp.int32, (G, BLK), 0)
    m = (bt_ref[...].reshape(1, BLK) == ids).astype(jnp.float32)
    acc_ref[...] += jnp.dot(m, x_ref[...], preferred_element_type=jnp.float32)

    @pl.when(i == NBLK - 1)
    def _():
        p = jnp.dot(acc_ref[...], w_ref[...], preferred_element_type=jnp.float32)
        o_ref[...] = jnp.maximum(p + b_ref[...], 0.0)


_tc_pool = pl.pallas_call(
    _tc_pool_body,
    grid=(NBLK,),
    in_specs=[
        pl.BlockSpec((1, 1, BLK), lambda i: (i, 0, 0)),
        pl.BlockSpec((BLK, D), lambda i: (i, 0)),
        pl.BlockSpec((D, D), lambda i: (0, 0)),
        pl.BlockSpec((1, D), lambda i: (0, 0)),
    ],
    out_specs=pl.BlockSpec((G, D), lambda i: (0, 0)),
    out_shape=jax.ShapeDtypeStruct((G, D), jnp.float32),
    scratch_shapes=[pltpu.VMEM((G, D), jnp.float32)],
)


def kernel(x, edge_index, batch,
           c1_w1, c1_b1, c1_g, c1_be, c1_w2, c1_b2,
           c2_w1, c2_b1, c2_g, c2_be, c2_w2, c2_b2,
           c3_w1, c3_b1, c3_g, c3_be, c3_w2, c3_b2,
           c4_w1, c4_b1, c4_g, c4_be, c4_w2, c4_b2,
           c5_w1, c5_b1, c5_g, c5_be, c5_w2, c5_b2,
           lin1_w, lin1_b):
    pad = EPW_PAD * NW - E
    pad_src = jnp.arange(pad, dtype=jnp.int32) % N
    pad_dst = N + jnp.arange(pad, dtype=jnp.int32) % (NPAD - N)
    src = jnp.concatenate([edge_index[0], pad_src])
    dst = jnp.concatenate([edge_index[1], pad_dst])
    src_r = src.reshape(NW, NCHUNK, CH)
    dst_r = dst.reshape(NW, NCHUNK, CH)
    zeros = jnp.zeros((NPAD, D), jnp.float32)
    batch_r = batch.reshape(NBLK, 1, BLK)

    inv = 1.0 / jnp.sqrt(jnp.float32(1.0 + 1e-5))
    layers = [
        (c1_w1, c1_b1, c1_g, c1_be, c1_w2, c1_b2),
        (c2_w1, c2_b1, c2_g, c2_be, c2_w2, c2_b2),
        (c3_w1, c3_b1, c3_g, c3_be, c3_w2, c3_b2),
        (c4_w1, c4_b1, c4_g, c4_be, c4_w2, c4_b2),
        (c5_w1, c5_b1, c5_g, c5_be, c5_w2, c5_b2),
    ]
    for w1, b1, g, be, w2, b2 in layers:
        part = _sc_scatter(x, src_r, dst_r, zeros)
        sc = (g * inv).reshape(1, D)
        sh = (b1 * g * inv + be).reshape(1, D)
        x = _tc_layer(x, part, part, w1, sc, sh, w2, b2.reshape(1, D))
    return _tc_pool(batch_r, x, lin1_w, lin1_b.reshape(1, D))
